# Initial kernel scaffold; baseline (speedup 1.0000x reference)
import jax
import jax.numpy as jnp
from jax.experimental import pallas as pl


def kernel(x, V, edge_index, gin_params, rho_params, pe_w, pe_b, sage_params, head_w, head_b):
    raise NotImplementedError("write your pallas kernel here")


# speedup vs baseline: 4.1180x; 4.1180x over previous
import functools

import jax
import jax.numpy as jnp
from jax import lax
from jax.experimental import pallas as pl
from jax.experimental.pallas import tpu as pltpu
from jax.experimental.pallas import tpu_sc as plsc

N = 10000
E = 160000
K = 8
CH = 128
HID = 120
PHI_OUT = 4
B = 512
RHO_OUT = 8

PW = 128
S = 16
NB = 400

NSC = 2
NT = 16
NR = 10240
TROWS = NR // NT

f32 = jnp.float32
i32 = jnp.int32


@functools.cache
def _mesh():
    return plsc.VectorSubcoreMesh(core_axis_name="c", subcore_axis_name="s")


def _sc_segsum_wide(table, srcg, dst, zeros):
    BK = 80
    EPT = E // NT
    NBATCH = EPT // BK
    SPC = S // NSC

    @functools.partial(
        pl.kernel,
        mesh=_mesh(),
        out_type=jax.ShapeDtypeStruct((S * NR, PW), f32),
        scratch_types=[
            pltpu.VMEM((BK,), i32),
            pltpu.VMEM((BK,), i32),
            pltpu.VMEM((BK, PW), f32),
            pltpu.VMEM_SHARED((NR, PW), f32),
            pltpu.SemaphoreType.DMA,
        ],
    )
    def k(table_hbm, srcg_hbm, dst_hbm, zeros_hbm, out_hbm,
          idx_v, dst_v, rows_v, acc_sh, sem):
        c = lax.axis_index("c")
        w = lax.axis_index("s")
        rbase = w * TROWS

        def stream_body(si, carry):
            s_chunk = c * SPC + si
            pltpu.sync_copy(zeros_hbm.at[pl.ds(0, TROWS)],
                            acc_sh.at[pl.ds(rbase, TROWS)])
            plsc.subcore_barrier()

            def batch_body(j, carry2):
                e0 = s_chunk * E + w * EPT + j * BK
                d0 = w * EPT + j * BK
                pltpu.sync_copy(srcg_hbm.at[pl.ds(e0, BK)], idx_v)
                pltpu.sync_copy(dst_hbm.at[pl.ds(d0, BK)], dst_v)
                pltpu.async_copy(table_hbm.at[idx_v], rows_v, sem).wait()
                pltpu.sync_copy(rows_v, acc_sh.at[dst_v], add=True)
                return carry2

            lax.fori_loop(0, NBATCH, batch_body, 0)
            plsc.subcore_barrier()
            pltpu.sync_copy(acc_sh.at[pl.ds(rbase, TROWS)],
                            out_hbm.at[pl.ds(s_chunk * NR + rbase, TROWS)])
            plsc.subcore_barrier()
            return carry

        lax.fori_loop(0, SPC, stream_body, 0)

    return k(table, srcg, dst, zeros)


def _sc_segsum_single(table, src, dst, zeros):
    BK = 40
    EPT = E // (NSC * NT)
    NBATCH = EPT // BK

    @functools.partial(
        pl.kernel,
        mesh=_mesh(),
        out_type=jax.ShapeDtypeStruct((NSC * NR, PW), f32),
        scratch_types=[
            pltpu.VMEM((BK,), i32),
            pltpu.VMEM((BK,), i32),
            pltpu.VMEM((BK, PW), f32),
            pltpu.VMEM_SHARED((NR, PW), f32),
            pltpu.SemaphoreType.DMA,
        ],
    )
    def k(table_hbm, src_hbm, dst_hbm, zeros_hbm, out_hbm,
          idx_v, dst_v, rows_v, acc_sh, sem):
        c = lax.axis_index("c")
        w = lax.axis_index("s")
        rbase = w * TROWS
        ebase = (c * NT + w) * EPT

        pltpu.sync_copy(zeros_hbm.at[pl.ds(0, TROWS)],
                        acc_sh.at[pl.ds(rbase, TROWS)])
        plsc.subcore_barrier()

        def batch_body(j, carry):
            e0 = ebase + j * BK
            pltpu.sync_copy(src_hbm.at[pl.ds(e0, BK)], idx_v)
            pltpu.sync_copy(dst_hbm.at[pl.ds(e0, BK)], dst_v)
            pltpu.async_copy(table_hbm.at[idx_v], rows_v, sem).wait()
            pltpu.sync_copy(rows_v, acc_sh.at[dst_v], add=True)
            return carry

        lax.fori_loop(0, NBATCH, batch_body, 0)
        plsc.subcore_barrier()
        pltpu.sync_copy(acc_sh.at[pl.ds(rbase, TROWS)],
                        out_hbm.at[pl.ds(c * NR + rbase, TROWS)])

    return k(table, src, dst, zeros)


def _dot(a, b):
    return jnp.dot(a, b, preferred_element_type=f32)


def _tc_gin_layer1(t0, a0a, a0b, w1row, b1, w2, b2):
    def body(t0_r, a0a_r, a0b_r, w1_r, b1_r, w2_r, b2_r, out_r):
        s = pl.program_id(0)
        kk = jnp.remainder(s, K)
        sign = jnp.where(s < K, 1.0, -1.0).astype(f32)
        zfull = t0_r[...] + a0a_r[...] + a0b_r[...]
        sel = (lax.broadcasted_iota(i32, (NB, PW), 1) == kk).astype(f32)
        z = jnp.sum(zfull * sel, axis=1, keepdims=True) * sign
        pre = z * w1_r[...] + b1_r[...]
        out_r[0] = _dot(jnp.maximum(pre, 0.0), w2_r[...]) + b2_r[...]

    return pl.pallas_call(
        body,
        grid=(S, N // NB),
        in_specs=[
            pl.BlockSpec((NB, PW), lambda s, n: (n, 0)),
            pl.BlockSpec((NB, PW), lambda s, n: (n, 0)),
            pl.BlockSpec((NB, PW), lambda s, n: (n, 0)),
            pl.BlockSpec((1, PW), lambda s, n: (0, 0)),
            pl.BlockSpec((1, PW), lambda s, n: (0, 0)),
            pl.BlockSpec((PW, PW), lambda s, n: (0, 0)),
            pl.BlockSpec((1, PW), lambda s, n: (0, 0)),
        ],
        out_specs=pl.BlockSpec((1, NB, PW), lambda s, n: (s, n, 0)),
        out_shape=jax.ShapeDtypeStruct((S, N, PW), f32),
    )(t0, a0a, a0b, w1row, b1, w2, b2)


def _tc_gin_layer(h, agg, w1, b1, w2, b2):
    def body(h_r, agg_r, w1_r, b1_r, w2_r, b2_r, out_r):
        z = h_r[0] + agg_r[0]
        pre = _dot(z, w1_r[...]) + b1_r[...]
        out_r[0] = _dot(jnp.maximum(pre, 0.0), w2_r[...]) + b2_r[...]

    return pl.pallas_call(
        body,
        grid=(S, N // NB),
        in_specs=[
            pl.BlockSpec((1, NB, PW), lambda s, n: (s, n, 0)),
            pl.BlockSpec((1, NB, PW), lambda s, n: (s, n, 0)),
            pl.BlockSpec((PW, PW), lambda s, n: (0, 0)),
            pl.BlockSpec((1, PW), lambda s, n: (0, 0)),
            pl.BlockSpec((PW, PW), lambda s, n: (0, 0)),
            pl.BlockSpec((1, PW), lambda s, n: (0, 0)),
        ],
        out_specs=pl.BlockSpec((1, NB, PW), lambda s, n: (s, n, 0)),
        out_shape=jax.ShapeDtypeStruct((S, N, PW), f32),
    )(h, agg, w1, b1, w2, b2)


def _tc_rho_pe(phi, x, g, b1, w2, b2, w3, b3, w4, b4, pew, peb):
    def body(phi_r, x_r, g_r, b1_r, w2_r, b2_r, w3_r, b3_r, w4_r, b4_r,
             pew_r, peb_r, out_r):
        t = b1_r[...]
        for s in range(S):
            t = t + _dot(phi_r[s], g_r[s])
        t = jnp.maximum(t, 0.0)
        t = jnp.maximum(_dot(t, w2_r[...]) + b2_r[...], 0.0)
        t = jnp.maximum(_dot(t, w3_r[...]) + b3_r[...], 0.0)
        t = _dot(t, w4_r[...]) + b4_r[...]
        out_r[...] = x_r[...] + _dot(t, pew_r[...]) + peb_r[...]

    wspec = pl.BlockSpec((PW, PW), lambda n: (0, 0))
    bspec = pl.BlockSpec((1, PW), lambda n: (0, 0))
    return pl.pallas_call(
        body,
        grid=(N // NB,),
        in_specs=[
            pl.BlockSpec((S, NB, PW), lambda n: (0, n, 0)),
            pl.BlockSpec((NB, PW), lambda n: (n, 0)),
            pl.BlockSpec((S, PW, PW), lambda n: (0, 0, 0)),
            bspec, wspec, bspec, wspec, bspec, wspec, bspec, wspec, bspec,
        ],
        out_specs=pl.BlockSpec((NB, PW), lambda n: (n, 0)),
        out_shape=jax.ShapeDtypeStruct((N, PW), f32),
    )(phi, x, g, b1, w2, b2, w3, b3, w4, b4, pew, peb)


def _tc_sage_layer(h, agga, aggb, a0a, a0b, ws, wn, b):
    def body(h_r, agga_r, aggb_r, a0a_r, a0b_r, ws_r, wn_r, b_r, out_r):
        degsel = (lax.broadcasted_iota(i32, (NB, PW), 1) == K).astype(f32)
        deg = jnp.sum((a0a_r[...] + a0b_r[...]) * degsel, axis=1,
                      keepdims=True)
        deg = jnp.maximum(deg, 1.0)
        agg = (agga_r[...] + aggb_r[...]) / deg
        pre = _dot(h_r[...], ws_r[...]) + _dot(agg, wn_r[...]) + b_r[...]
        out_r[...] = jnp.maximum(pre, 0.0)

    nspec = pl.BlockSpec((NB, PW), lambda n: (n, 0))
    wspec = pl.BlockSpec((PW, PW), lambda n: (0, 0))
    return pl.pallas_call(
        body,
        grid=(N // NB,),
        in_specs=[nspec, nspec, nspec, nspec, nspec, wspec, wspec,
                  pl.BlockSpec((1, PW), lambda n: (0, 0))],
        out_specs=nspec,
        out_shape=jax.ShapeDtypeStruct((N, PW), f32),
    )(h, agga, aggb, a0a, a0b, ws, wn, b)


def _tc_head(h, w, bias):
    def body(h_r, w_r, b_r, out_r):
        out_r[...] = _dot(h_r[...], w_r[...]) + b_r[...]

    return pl.pallas_call(
        body,
        grid=(1,),
        in_specs=[
            pl.BlockSpec((B, PW), lambda i: (0, 0)),
            pl.BlockSpec((PW, PW), lambda i: (0, 0)),
            pl.BlockSpec((1, PW), lambda i: (0, 0)),
        ],
        out_specs=pl.BlockSpec((B, PW), lambda i: (0, 0)),
        out_shape=jax.ShapeDtypeStruct((B, PW), f32),
    )(h, w, bias)


def _padw(w):
    out = jnp.zeros((PW, PW), f32)
    return out.at[: w.shape[0], : w.shape[1]].set(w)


def _padb(bvec):
    out = jnp.zeros((1, PW), f32)
    return out.at[0, : bvec.shape[0]].set(bvec)


def kernel(x, V, edge_index, gin_params, rho_params, pe_w, pe_b,
           sage_params, head_w, head_b):
    src = edge_index[0].astype(i32)
    dst = edge_index[1].astype(i32)
    zeros = jnp.zeros((TROWS, PW), f32)

    t0 = jnp.zeros((N, PW), f32).at[:, :K].set(V).at[:, K].set(1.0)

    srcg = (jnp.arange(S, dtype=i32)[:, None] * N + src[None, :]).reshape(-1)

    a0 = _sc_segsum_single(t0, src, dst, zeros)
    a0a, a0b = a0[:N], a0[NR:NR + N]

    (w1_1, b1_1, w2_1, b2_1) = gin_params[0]
    h = _tc_gin_layer1(t0, a0a, a0b,
                       _padb(w1_1[0]), _padb(b1_1), _padw(w2_1), _padb(b2_1))

    for (w1, b1, w2, b2) in gin_params[1:]:
        agg = _sc_segsum_wide(h.reshape(S * N, PW), srcg, dst, zeros)
        h = _tc_gin_layer(h, agg.reshape(S, NR, PW)[:, :N],
                          _padw(w1), _padb(b1), _padw(w2), _padb(b2))

    w_rho1 = rho_params[0][0]
    g = jnp.zeros((S, PW, PW), f32)
    for s in range(S):
        kk = s % K
        g = g.at[s, :PHI_OUT, :HID].set(
            w_rho1[kk * PHI_OUT:(kk + 1) * PHI_OUT, :])
    henc = _tc_rho_pe(
        h, x, g, _padb(rho_params[0][1]),
        _padw(rho_params[1][0]), _padb(rho_params[1][1]),
        _padw(rho_params[2][0]), _padb(rho_params[2][1]),
        _padw(rho_params[3][0]), _padb(rho_params[3][1]),
        _padw(pe_w), _padb(pe_b))

    for (ws, wn, bb) in sage_params:
        ap = _sc_segsum_single(henc, src, dst, zeros)
        henc = _tc_sage_layer(henc, ap[:N], ap[NR:NR + N], a0a, a0b,
                              _padw(ws), _padw(wn), _padb(bb))

    out = _tc_head(henc[:B], _padw(head_w), _padb(head_b))
    return out[:, :1]


# speedup vs baseline: 8.2991x; 2.0153x over previous
import functools

import jax
import jax.numpy as jnp
from jax import lax
from jax.experimental import pallas as pl
from jax.experimental.pallas import tpu as pltpu
from jax.experimental.pallas import tpu_sc as plsc

N = 10000
E = 160000
K = 8
CH = 128
HID = 120
PHI_OUT = 4
B = 512
RHO_OUT = 8

PW = 128
S = 16
NB = 400

NSC = 2
NT = 16
NR = 10240
TROWS = NR // NT

f32 = jnp.float32
i32 = jnp.int32


@functools.cache
def _mesh():
    return plsc.VectorSubcoreMesh(core_axis_name="c", subcore_axis_name="s")


BK = 125
NBW = (E // NT) // BK
NB1 = (E // (NSC * NT)) // BK


def _sc_segsum_wide(table, srcg, dst, zeros):
    SPC = S // NSC

    @functools.partial(
        pl.kernel,
        mesh=_mesh(),
        out_type=jax.ShapeDtypeStruct((S * NR, PW), f32),
        scratch_types=[
            pltpu.VMEM((NBW // 2, BK), i32),
            pltpu.VMEM((NBW, BK), i32),
            pltpu.VMEM((BK, PW), f32),
            pltpu.VMEM((BK, PW), f32),
            pltpu.VMEM_SHARED((NR, PW), f32),
            pltpu.SemaphoreType.DMA,
            pltpu.SemaphoreType.DMA,
        ],
    )
    def k(table_hbm, srcg_hbm, dst_hbm, zeros_hbm, out_hbm,
          src_blk, dst_blk, rows_a, rows_b, acc_sh, sem_a, sem_b):
        c = lax.axis_index("c")
        w = lax.axis_index("s")
        rbase = w * TROWS
        NH = NBW // 2
        pltpu.sync_copy(dst_hbm.at[pl.ds(w * NBW, NBW)], dst_blk)

        def stream_body(si, carry):
            s_chunk = c * SPC + si
            pltpu.sync_copy(zeros_hbm.at[pl.ds(0, TROWS)],
                            acc_sh.at[pl.ds(rbase, TROWS)])
            plsc.subcore_barrier()

            def half_body(hi, carry1):
                pltpu.sync_copy(
                    srcg_hbm.at[pl.ds((s_chunk * NT + w) * NBW + hi * NH,
                                      NH)], src_blk)
                pltpu.async_copy(table_hbm.at[src_blk.at[0]], rows_a, sem_a)

                def batch_body(j2, carry2):
                    b = 2 * j2
                    d = hi * NH + b
                    pltpu.async_copy(table_hbm.at[src_blk.at[b + 1]],
                                     rows_b, sem_b)
                    pltpu.make_async_copy(table_hbm.at[src_blk.at[b]],
                                          rows_a, sem_a).wait()
                    pltpu.sync_copy(rows_a, acc_sh.at[dst_blk.at[d]],
                                    add=True)

                    @pl.when(b + 2 < NH)
                    def _():
                        pltpu.async_copy(table_hbm.at[src_blk.at[b + 2]],
                                         rows_a, sem_a)

                    pltpu.make_async_copy(table_hbm.at[src_blk.at[b + 1]],
                                          rows_b, sem_b).wait()
                    pltpu.sync_copy(rows_b, acc_sh.at[dst_blk.at[d + 1]],
                                    add=True)
                    return carry2

                lax.fori_loop(0, NH // 2, batch_body, 0)
                return carry1

            lax.fori_loop(0, 2, half_body, 0)
            plsc.subcore_barrier()
            pltpu.sync_copy(acc_sh.at[pl.ds(rbase, TROWS)],
                            out_hbm.at[pl.ds(s_chunk * NR + rbase, TROWS)])
            plsc.subcore_barrier()
            return carry

        lax.fori_loop(0, SPC, stream_body, 0)

    return k(table, srcg, dst, zeros)


def _sc_segsum_single(table, src, dst, zeros):
    @functools.partial(
        pl.kernel,
        mesh=_mesh(),
        out_type=jax.ShapeDtypeStruct((NSC * NR, PW), f32),
        scratch_types=[
            pltpu.VMEM((NB1, BK), i32),
            pltpu.VMEM((NB1, BK), i32),
            pltpu.VMEM((BK, PW), f32),
            pltpu.VMEM((BK, PW), f32),
            pltpu.VMEM_SHARED((NR, PW), f32),
            pltpu.SemaphoreType.DMA,
            pltpu.SemaphoreType.DMA,
        ],
    )
    def k(table_hbm, src_hbm, dst_hbm, zeros_hbm, out_hbm,
          src_blk, dst_blk, rows_a, rows_b, acc_sh, sem_a, sem_b):
        c = lax.axis_index("c")
        w = lax.axis_index("s")
        rbase = w * TROWS
        bbase = (c * NT + w) * NB1

        pltpu.sync_copy(src_hbm.at[pl.ds(bbase, NB1)], src_blk)
        pltpu.sync_copy(dst_hbm.at[pl.ds(bbase, NB1)], dst_blk)
        pltpu.sync_copy(zeros_hbm.at[pl.ds(0, TROWS)],
                        acc_sh.at[pl.ds(rbase, TROWS)])
        plsc.subcore_barrier()

        pltpu.async_copy(table_hbm.at[src_blk.at[0]], rows_a, sem_a)

        def batch_body(j2, carry):
            b = 2 * j2
            pltpu.async_copy(table_hbm.at[src_blk.at[b + 1]], rows_b, sem_b)
            pltpu.make_async_copy(table_hbm.at[src_blk.at[b]],
                                  rows_a, sem_a).wait()
            pltpu.sync_copy(rows_a, acc_sh.at[dst_blk.at[b]], add=True)

            @pl.when(b + 2 < NB1)
            def _():
                pltpu.async_copy(table_hbm.at[src_blk.at[b + 2]],
                                 rows_a, sem_a)

            pltpu.make_async_copy(table_hbm.at[src_blk.at[b + 1]],
                                  rows_b, sem_b).wait()
            pltpu.sync_copy(rows_b, acc_sh.at[dst_blk.at[b + 1]], add=True)
            return carry

        lax.fori_loop(0, NB1 // 2, batch_body, 0)
        plsc.subcore_barrier()
        pltpu.sync_copy(acc_sh.at[pl.ds(rbase, TROWS)],
                        out_hbm.at[pl.ds(c * NR + rbase, TROWS)])

    return k(table, src, dst, zeros)


bf16 = jnp.bfloat16


def _split3(a):
    a1 = a.astype(bf16)
    r = a - a1.astype(f32)
    a2 = r.astype(bf16)
    a3 = (r - a2.astype(f32)).astype(bf16)
    return a1, a2, a3


def _dot(a, b):
    a1, a2, a3 = _split3(a)
    b1, b2, b3 = _split3(b)
    def d(p, q):
        return jnp.dot(p, q, preferred_element_type=f32)
    return ((d(a3, b1) + d(a2, b2) + d(a1, b3))
            + (d(a2, b1) + d(a1, b2))) + d(a1, b1)


def _tc_gin_layer1(t0, a0a, a0b, w1row, b1, w2, b2):
    def body(t0_r, a0a_r, a0b_r, w1_r, b1_r, w2_r, b2_r, out_r):
        s = pl.program_id(0)
        kk = jnp.remainder(s, K)
        sign = jnp.where(s < K, 1.0, -1.0).astype(f32)
        zfull = t0_r[...] + a0a_r[...] + a0b_r[...]
        sel = (lax.broadcasted_iota(i32, (NB, PW), 1) == kk).astype(f32)
        z = jnp.sum(zfull * sel, axis=1, keepdims=True) * sign
        pre = z * w1_r[...] + b1_r[...]
        out_r[0] = _dot(jnp.maximum(pre, 0.0), w2_r[...]) + b2_r[...]

    return pl.pallas_call(
        body,
        grid=(S, N // NB),
        in_specs=[
            pl.BlockSpec((NB, PW), lambda s, n: (n, 0)),
            pl.BlockSpec((NB, PW), lambda s, n: (n, 0)),
            pl.BlockSpec((NB, PW), lambda s, n: (n, 0)),
            pl.BlockSpec((1, PW), lambda s, n: (0, 0)),
            pl.BlockSpec((1, PW), lambda s, n: (0, 0)),
            pl.BlockSpec((PW, PW), lambda s, n: (0, 0)),
            pl.BlockSpec((1, PW), lambda s, n: (0, 0)),
        ],
        out_specs=pl.BlockSpec((1, NB, PW), lambda s, n: (s, n, 0)),
        out_shape=jax.ShapeDtypeStruct((S, N, PW), f32),
    )(t0, a0a, a0b, w1row, b1, w2, b2)


def _tc_gin_layer(h, agg, w1, b1, w2, b2):
    def body(h_r, agg_r, w1_r, b1_r, w2_r, b2_r, out_r):
        z = h_r[0] + agg_r[0]
        pre = _dot(z, w1_r[...]) + b1_r[...]
        out_r[0] = _dot(jnp.maximum(pre, 0.0), w2_r[...]) + b2_r[...]

    return pl.pallas_call(
        body,
        grid=(S, N // NB),
        in_specs=[
            pl.BlockSpec((1, NB, PW), lambda s, n: (s, n, 0)),
            pl.BlockSpec((1, NB, PW), lambda s, n: (s, n, 0)),
            pl.BlockSpec((PW, PW), lambda s, n: (0, 0)),
            pl.BlockSpec((1, PW), lambda s, n: (0, 0)),
            pl.BlockSpec((PW, PW), lambda s, n: (0, 0)),
            pl.BlockSpec((1, PW), lambda s, n: (0, 0)),
        ],
        out_specs=pl.BlockSpec((1, NB, PW), lambda s, n: (s, n, 0)),
        out_shape=jax.ShapeDtypeStruct((S, N, PW), f32),
    )(h, agg, w1, b1, w2, b2)


def _tc_rho_pe(phi, x, g, b1, w2, b2, w3, b3, w4, b4, pew, peb):
    def body(phi_r, x_r, g_r, b1_r, w2_r, b2_r, w3_r, b3_r, w4_r, b4_r,
             pew_r, peb_r, out_r):
        t = b1_r[...]
        for kk in range(K):
            t = t + _dot(phi_r[kk] + phi_r[kk + K], g_r[kk])
        t = jnp.maximum(t, 0.0)
        t = jnp.maximum(_dot(t, w2_r[...]) + b2_r[...], 0.0)
        t = jnp.maximum(_dot(t, w3_r[...]) + b3_r[...], 0.0)
        t = _dot(t, w4_r[...]) + b4_r[...]
        out_r[...] = x_r[...] + _dot(t, pew_r[...]) + peb_r[...]

    wspec = pl.BlockSpec((PW, PW), lambda n: (0, 0))
    bspec = pl.BlockSpec((1, PW), lambda n: (0, 0))
    return pl.pallas_call(
        body,
        grid=(N // NB,),
        in_specs=[
            pl.BlockSpec((S, NB, PW), lambda n: (0, n, 0)),
            pl.BlockSpec((NB, PW), lambda n: (n, 0)),
            pl.BlockSpec((K, PW, PW), lambda n: (0, 0, 0)),
            bspec, wspec, bspec, wspec, bspec, wspec, bspec, wspec, bspec,
        ],
        out_specs=pl.BlockSpec((NB, PW), lambda n: (n, 0)),
        out_shape=jax.ShapeDtypeStruct((N, PW), f32),
    )(phi, x, g, b1, w2, b2, w3, b3, w4, b4, pew, peb)


def _tc_sage_layer(h, agga, aggb, a0a, a0b, ws, wn, b):
    def body(h_r, agga_r, aggb_r, a0a_r, a0b_r, ws_r, wn_r, b_r, out_r):
        degsel = (lax.broadcasted_iota(i32, (NB, PW), 1) == K).astype(f32)
        deg = jnp.sum((a0a_r[...] + a0b_r[...]) * degsel, axis=1,
                      keepdims=True)
        deg = jnp.maximum(deg, 1.0)
        agg = (agga_r[...] + aggb_r[...]) / deg
        pre = _dot(h_r[...], ws_r[...]) + _dot(agg, wn_r[...]) + b_r[...]
        out_r[...] = jnp.maximum(pre, 0.0)

    nspec = pl.BlockSpec((NB, PW), lambda n: (n, 0))
    wspec = pl.BlockSpec((PW, PW), lambda n: (0, 0))
    return pl.pallas_call(
        body,
        grid=(N // NB,),
        in_specs=[nspec, nspec, nspec, nspec, nspec, wspec, wspec,
                  pl.BlockSpec((1, PW), lambda n: (0, 0))],
        out_specs=nspec,
        out_shape=jax.ShapeDtypeStruct((N, PW), f32),
    )(h, agga, aggb, a0a, a0b, ws, wn, b)


def _tc_head(h, w, bias):
    def body(h_r, w_r, b_r, out_r):
        out_r[...] = _dot(h_r[...], w_r[...]) + b_r[...]

    return pl.pallas_call(
        body,
        grid=(1,),
        in_specs=[
            pl.BlockSpec((B, PW), lambda i: (0, 0)),
            pl.BlockSpec((PW, PW), lambda i: (0, 0)),
            pl.BlockSpec((1, PW), lambda i: (0, 0)),
        ],
        out_specs=pl.BlockSpec((B, PW), lambda i: (0, 0)),
        out_shape=jax.ShapeDtypeStruct((B, PW), f32),
    )(h, w, bias)


def _padw(w):
    out = jnp.zeros((PW, PW), f32)
    return out.at[: w.shape[0], : w.shape[1]].set(w)


def _padb(bvec):
    out = jnp.zeros((1, PW), f32)
    return out.at[0, : bvec.shape[0]].set(bvec)


def kernel(x, V, edge_index, gin_params, rho_params, pe_w, pe_b,
           sage_params, head_w, head_b):
    src = edge_index[0].astype(i32)
    dst = edge_index[1].astype(i32)
    src2 = src.reshape(NSC * NT * NB1, BK)
    dst2 = dst.reshape(NT * NBW, BK)
    zeros = jnp.zeros((TROWS, PW), f32)

    t0 = jnp.zeros((N, PW), f32).at[:, :K].set(V).at[:, K].set(1.0)

    srcg = (jnp.arange(S, dtype=i32)[:, None] * N
            + src[None, :]).reshape(S * NT * NBW, BK)

    a0 = _sc_segsum_single(t0, src2, dst2, zeros)
    a0a, a0b = a0[:N], a0[NR:NR + N]

    (w1_1, b1_1, w2_1, b2_1) = gin_params[0]
    h = _tc_gin_layer1(t0, a0a, a0b,
                       _padb(w1_1[0]), _padb(b1_1), _padw(w2_1), _padb(b2_1))

    for (w1, b1, w2, b2) in gin_params[1:]:
        agg = _sc_segsum_wide(h.reshape(S * N, PW), srcg, dst2, zeros)
        h = _tc_gin_layer(h, agg.reshape(S, NR, PW)[:, :N],
                          _padw(w1), _padb(b1), _padw(w2), _padb(b2))

    w_rho1 = rho_params[0][0]
    g = jnp.zeros((K, PW, PW), f32)
    for kk in range(K):
        g = g.at[kk, :PHI_OUT, :HID].set(
            w_rho1[kk * PHI_OUT:(kk + 1) * PHI_OUT, :])
    henc = _tc_rho_pe(
        h, x, g, _padb(rho_params[0][1]),
        _padw(rho_params[1][0]), _padb(rho_params[1][1]),
        _padw(rho_params[2][0]), _padb(rho_params[2][1]),
        _padw(rho_params[3][0]), _padb(rho_params[3][1]),
        _padw(pe_w), _padb(pe_b))

    for (ws, wn, bb) in sage_params:
        ap = _sc_segsum_single(henc, src2, dst2, zeros)
        henc = _tc_sage_layer(henc, ap[:N], ap[NR:NR + N], a0a, a0b,
                              _padw(ws), _padw(wn), _padb(bb))

    out = _tc_head(henc[:B], _padw(head_w), _padb(head_b))
    return out[:, :1]


# speedup vs baseline: 9.3485x; 1.1265x over previous
import functools

import jax
import jax.numpy as jnp
from jax import lax
from jax.experimental import pallas as pl
from jax.experimental.pallas import tpu as pltpu
from jax.experimental.pallas import tpu_sc as plsc

N = 10000
E = 160000
K = 8
CH = 128
HID = 120
PHI_OUT = 4
B = 512
RHO_OUT = 8

PW = 128
S = 16
NB = 400

NSC = 2
NT = 16
NR = 10240
TROWS = NR // NT

f32 = jnp.float32
i32 = jnp.int32


@functools.cache
def _mesh():
    return plsc.VectorSubcoreMesh(core_axis_name="c", subcore_axis_name="s")


BK = 125
NBW = (E // NT) // BK
NB1 = (E // (NSC * NT)) // BK


def _sc_segsum_wide(table, srcg, dst, zeros):
    SPC = S // NSC

    @functools.partial(
        pl.kernel,
        mesh=_mesh(),
        out_type=jax.ShapeDtypeStruct((S * NR, PW), f32),
        scratch_types=[
            pltpu.VMEM((NBW // 2, BK), i32),
            pltpu.VMEM((NBW, BK), i32),
            pltpu.VMEM((BK, PW), f32),
            pltpu.VMEM((BK, PW), f32),
            pltpu.VMEM_SHARED((NR, PW), f32),
            pltpu.SemaphoreType.DMA,
            pltpu.SemaphoreType.DMA,
        ],
    )
    def k(table_hbm, srcg_hbm, dst_hbm, zeros_hbm, out_hbm,
          src_blk, dst_blk, rows_a, rows_b, acc_sh, sem_a, sem_b):
        c = lax.axis_index("c")
        w = lax.axis_index("s")
        rbase = w * TROWS
        NH = NBW // 2
        pltpu.sync_copy(dst_hbm.at[pl.ds(w * NBW, NBW)], dst_blk)

        def stream_body(si, carry):
            s_chunk = c * SPC + si
            pltpu.sync_copy(zeros_hbm.at[pl.ds(0, TROWS)],
                            acc_sh.at[pl.ds(rbase, TROWS)])
            plsc.subcore_barrier()

            def half_body(hi, carry1):
                pltpu.sync_copy(
                    srcg_hbm.at[pl.ds((s_chunk * NT + w) * NBW + hi * NH,
                                      NH)], src_blk)
                pltpu.async_copy(table_hbm.at[src_blk.at[0]], rows_a, sem_a)

                def batch_body(j2, carry2):
                    b = 2 * j2
                    d = hi * NH + b
                    pltpu.async_copy(table_hbm.at[src_blk.at[b + 1]],
                                     rows_b, sem_b)
                    pltpu.make_async_copy(table_hbm.at[src_blk.at[b]],
                                          rows_a, sem_a).wait()
                    pltpu.sync_copy(rows_a, acc_sh.at[dst_blk.at[d]],
                                    add=True)

                    @pl.when(b + 2 < NH)
                    def _():
                        pltpu.async_copy(table_hbm.at[src_blk.at[b + 2]],
                                         rows_a, sem_a)

                    pltpu.make_async_copy(table_hbm.at[src_blk.at[b + 1]],
                                          rows_b, sem_b).wait()
                    pltpu.sync_copy(rows_b, acc_sh.at[dst_blk.at[d + 1]],
                                    add=True)
                    return carry2

                lax.fori_loop(0, NH // 2, batch_body, 0)
                return carry1

            lax.fori_loop(0, 2, half_body, 0)
            plsc.subcore_barrier()
            pltpu.sync_copy(acc_sh.at[pl.ds(rbase, TROWS)],
                            out_hbm.at[pl.ds(s_chunk * NR + rbase, TROWS)])
            plsc.subcore_barrier()
            return carry

        lax.fori_loop(0, SPC, stream_body, 0)

    return k(table, srcg, dst, zeros)


def _sc_segsum_single(table, src, dst, zeros):
    @functools.partial(
        pl.kernel,
        mesh=_mesh(),
        out_type=jax.ShapeDtypeStruct((NSC * NR, PW), f32),
        scratch_types=[
            pltpu.VMEM((NB1, BK), i32),
            pltpu.VMEM((NB1, BK), i32),
            pltpu.VMEM((BK, PW), f32),
            pltpu.VMEM((BK, PW), f32),
            pltpu.VMEM_SHARED((NR, PW), f32),
            pltpu.SemaphoreType.DMA,
            pltpu.SemaphoreType.DMA,
        ],
    )
    def k(table_hbm, src_hbm, dst_hbm, zeros_hbm, out_hbm,
          src_blk, dst_blk, rows_a, rows_b, acc_sh, sem_a, sem_b):
        c = lax.axis_index("c")
        w = lax.axis_index("s")
        rbase = w * TROWS
        bbase = (c * NT + w) * NB1

        pltpu.sync_copy(src_hbm.at[pl.ds(bbase, NB1)], src_blk)
        pltpu.sync_copy(dst_hbm.at[pl.ds(bbase, NB1)], dst_blk)
        pltpu.sync_copy(zeros_hbm.at[pl.ds(0, TROWS)],
                        acc_sh.at[pl.ds(rbase, TROWS)])
        plsc.subcore_barrier()

        pltpu.async_copy(table_hbm.at[src_blk.at[0]], rows_a, sem_a)

        def batch_body(j2, carry):
            b = 2 * j2
            pltpu.async_copy(table_hbm.at[src_blk.at[b + 1]], rows_b, sem_b)
            pltpu.make_async_copy(table_hbm.at[src_blk.at[b]],
                                  rows_a, sem_a).wait()
            pltpu.sync_copy(rows_a, acc_sh.at[dst_blk.at[b]], add=True)

            @pl.when(b + 2 < NB1)
            def _():
                pltpu.async_copy(table_hbm.at[src_blk.at[b + 2]],
                                 rows_a, sem_a)

            pltpu.make_async_copy(table_hbm.at[src_blk.at[b + 1]],
                                  rows_b, sem_b).wait()
            pltpu.sync_copy(rows_b, acc_sh.at[dst_blk.at[b + 1]], add=True)
            return carry

        lax.fori_loop(0, NB1 // 2, batch_body, 0)
        plsc.subcore_barrier()
        pltpu.sync_copy(acc_sh.at[pl.ds(rbase, TROWS)],
                        out_hbm.at[pl.ds(c * NR + rbase, TROWS)])

    return k(table, src, dst, zeros)


def _dot(a, b):
    return jnp.dot(a, b, preferred_element_type=f32)


def _tc_gin_layer1(t0, a0a, a0b, w1row, b1, w2, b2):
    def body(t0_r, a0a_r, a0b_r, w1_r, b1_r, w2_r, b2_r, out_r):
        s = pl.program_id(0)
        kk = jnp.remainder(s, K)
        sign = jnp.where(s < K, 1.0, -1.0).astype(f32)
        zfull = t0_r[...] + a0a_r[...] + a0b_r[...]
        sel = (lax.broadcasted_iota(i32, (NB, PW), 1) == kk).astype(f32)
        z = jnp.sum(zfull * sel, axis=1, keepdims=True) * sign
        pre = z * w1_r[...] + b1_r[...]
        out_r[0] = _dot(jnp.maximum(pre, 0.0), w2_r[...]) + b2_r[...]

    return pl.pallas_call(
        body,
        grid=(S, N // NB),
        in_specs=[
            pl.BlockSpec((NB, PW), lambda s, n: (n, 0)),
            pl.BlockSpec((NB, PW), lambda s, n: (n, 0)),
            pl.BlockSpec((NB, PW), lambda s, n: (n, 0)),
            pl.BlockSpec((1, PW), lambda s, n: (0, 0)),
            pl.BlockSpec((1, PW), lambda s, n: (0, 0)),
            pl.BlockSpec((PW, PW), lambda s, n: (0, 0)),
            pl.BlockSpec((1, PW), lambda s, n: (0, 0)),
        ],
        out_specs=pl.BlockSpec((1, NB, PW), lambda s, n: (s, n, 0)),
        out_shape=jax.ShapeDtypeStruct((S, N, PW), f32),
    )(t0, a0a, a0b, w1row, b1, w2, b2)


def _tc_gin_layer(h, agg, w1, b1, w2, b2):
    def body(h_r, agg_r, w1_r, b1_r, w2_r, b2_r, out_r):
        z = h_r[0] + agg_r[0]
        pre = _dot(z, w1_r[...]) + b1_r[...]
        out_r[0] = _dot(jnp.maximum(pre, 0.0), w2_r[...]) + b2_r[...]

    return pl.pallas_call(
        body,
        grid=(S, N // NB),
        in_specs=[
            pl.BlockSpec((1, NB, PW), lambda s, n: (s, n, 0)),
            pl.BlockSpec((1, NB, PW), lambda s, n: (s, n, 0)),
            pl.BlockSpec((PW, PW), lambda s, n: (0, 0)),
            pl.BlockSpec((1, PW), lambda s, n: (0, 0)),
            pl.BlockSpec((PW, PW), lambda s, n: (0, 0)),
            pl.BlockSpec((1, PW), lambda s, n: (0, 0)),
        ],
        out_specs=pl.BlockSpec((1, NB, PW), lambda s, n: (s, n, 0)),
        out_shape=jax.ShapeDtypeStruct((S, N, PW), f32),
    )(h, agg, w1, b1, w2, b2)


def _tc_rho_pe(phi, x, g, b1, w2, b2, w3, b3, w4, b4, pew, peb):
    def body(phi_r, x_r, g_r, b1_r, w2_r, b2_r, w3_r, b3_r, w4_r, b4_r,
             pew_r, peb_r, out_r):
        t = b1_r[...]
        for kk in range(K):
            t = t + _dot(phi_r[kk] + phi_r[kk + K], g_r[kk])
        t = jnp.maximum(t, 0.0)
        t = jnp.maximum(_dot(t, w2_r[...]) + b2_r[...], 0.0)
        t = jnp.maximum(_dot(t, w3_r[...]) + b3_r[...], 0.0)
        t = _dot(t, w4_r[...]) + b4_r[...]
        out_r[...] = x_r[...] + _dot(t, pew_r[...]) + peb_r[...]

    wspec = pl.BlockSpec((PW, PW), lambda n: (0, 0))
    bspec = pl.BlockSpec((1, PW), lambda n: (0, 0))
    return pl.pallas_call(
        body,
        grid=(N // NB,),
        in_specs=[
            pl.BlockSpec((S, NB, PW), lambda n: (0, n, 0)),
            pl.BlockSpec((NB, PW), lambda n: (n, 0)),
            pl.BlockSpec((K, PW, PW), lambda n: (0, 0, 0)),
            bspec, wspec, bspec, wspec, bspec, wspec, bspec, wspec, bspec,
        ],
        out_specs=pl.BlockSpec((NB, PW), lambda n: (n, 0)),
        out_shape=jax.ShapeDtypeStruct((N, PW), f32),
    )(phi, x, g, b1, w2, b2, w3, b3, w4, b4, pew, peb)


def _tc_sage_layer(h, agga, aggb, a0a, a0b, ws, wn, b):
    def body(h_r, agga_r, aggb_r, a0a_r, a0b_r, ws_r, wn_r, b_r, out_r):
        degsel = (lax.broadcasted_iota(i32, (NB, PW), 1) == K).astype(f32)
        deg = jnp.sum((a0a_r[...] + a0b_r[...]) * degsel, axis=1,
                      keepdims=True)
        deg = jnp.maximum(deg, 1.0)
        agg = (agga_r[...] + aggb_r[...]) / deg
        pre = _dot(h_r[...], ws_r[...]) + _dot(agg, wn_r[...]) + b_r[...]
        out_r[...] = jnp.maximum(pre, 0.0)

    nspec = pl.BlockSpec((NB, PW), lambda n: (n, 0))
    wspec = pl.BlockSpec((PW, PW), lambda n: (0, 0))
    return pl.pallas_call(
        body,
        grid=(N // NB,),
        in_specs=[nspec, nspec, nspec, nspec, nspec, wspec, wspec,
                  pl.BlockSpec((1, PW), lambda n: (0, 0))],
        out_specs=nspec,
        out_shape=jax.ShapeDtypeStruct((N, PW), f32),
    )(h, agga, aggb, a0a, a0b, ws, wn, b)


def _tc_head(h, w, bias):
    def body(h_r, w_r, b_r, out_r):
        out_r[...] = _dot(h_r[...], w_r[...]) + b_r[...]

    return pl.pallas_call(
        body,
        grid=(1,),
        in_specs=[
            pl.BlockSpec((B, PW), lambda i: (0, 0)),
            pl.BlockSpec((PW, PW), lambda i: (0, 0)),
            pl.BlockSpec((1, PW), lambda i: (0, 0)),
        ],
        out_specs=pl.BlockSpec((B, PW), lambda i: (0, 0)),
        out_shape=jax.ShapeDtypeStruct((B, PW), f32),
    )(h, w, bias)


def _padw(w):
    out = jnp.zeros((PW, PW), f32)
    return out.at[: w.shape[0], : w.shape[1]].set(w)


def _padb(bvec):
    out = jnp.zeros((1, PW), f32)
    return out.at[0, : bvec.shape[0]].set(bvec)


def kernel(x, V, edge_index, gin_params, rho_params, pe_w, pe_b,
           sage_params, head_w, head_b):
    src = edge_index[0].astype(i32)
    dst = edge_index[1].astype(i32)
    src2 = src.reshape(NSC * NT * NB1, BK)
    dst2 = dst.reshape(NT * NBW, BK)
    zeros = jnp.zeros((TROWS, PW), f32)

    t0 = jnp.zeros((N, PW), f32).at[:, :K].set(V).at[:, K].set(1.0)

    srcg = (jnp.arange(S, dtype=i32)[:, None] * N
            + src[None, :]).reshape(S * NT * NBW, BK)

    a0 = _sc_segsum_single(t0, src2, dst2, zeros)
    a0a, a0b = a0[:N], a0[NR:NR + N]

    (w1_1, b1_1, w2_1, b2_1) = gin_params[0]
    h = _tc_gin_layer1(t0, a0a, a0b,
                       _padb(w1_1[0]), _padb(b1_1), _padw(w2_1), _padb(b2_1))

    for (w1, b1, w2, b2) in gin_params[1:]:
        agg = _sc_segsum_wide(h.reshape(S * N, PW), srcg, dst2, zeros)
        h = _tc_gin_layer(h, agg.reshape(S, NR, PW)[:, :N],
                          _padw(w1), _padb(b1), _padw(w2), _padb(b2))

    w_rho1 = rho_params[0][0]
    g = jnp.zeros((K, PW, PW), f32)
    for kk in range(K):
        g = g.at[kk, :PHI_OUT, :HID].set(
            w_rho1[kk * PHI_OUT:(kk + 1) * PHI_OUT, :])
    henc = _tc_rho_pe(
        h, x, g, _padb(rho_params[0][1]),
        _padw(rho_params[1][0]), _padb(rho_params[1][1]),
        _padw(rho_params[2][0]), _padb(rho_params[2][1]),
        _padw(rho_params[3][0]), _padb(rho_params[3][1]),
        _padw(pe_w), _padb(pe_b))

    for (ws, wn, bb) in sage_params:
        ap = _sc_segsum_single(henc, src2, dst2, zeros)
        henc = _tc_sage_layer(henc, ap[:N], ap[NR:NR + N], a0a, a0b,
                              _padw(ws), _padw(wn), _padb(bb))

    out = _tc_head(henc[:B], _padw(head_w), _padb(head_b))
    return out[:, :1]
